# baseline (speedup 1.0000x reference)
import functools

import jax
import jax.numpy as jnp
from jax.experimental import pallas as pl
from jax.experimental.pallas import tpu as pltpu


def _fused_body(adj_ref, xa0_ref, xs_ref, ws_ref, wn_ref, b_ref, out_ref,
                h_s, *, feat, bm):
    l = pl.program_id(0)
    i = pl.program_id(1)
    a = adj_ref[...].astype(jnp.bfloat16)
    base = pl.multiple_of(i * bm, bm)

    def _epilogue(prod, xs):
        s = prod[:, :feat]
        deg = jnp.clip(prod[:, feat:feat + 1], 1e-6, None)
        neigh = s / deg
        return (jnp.dot(xs, ws_ref[0], preferred_element_type=jnp.float32,
                        precision=jax.lax.Precision.HIGHEST)
                + jnp.dot(neigh, wn_ref[0],
                          preferred_element_type=jnp.float32,
                          precision=jax.lax.Precision.HIGHEST)
                + b_ref[0])

    @pl.when(l == 0)
    def _layer1():
        prod = jnp.dot(a, xa0_ref[...], preferred_element_type=jnp.float32)
        h = jnp.maximum(_epilogue(prod, xs_ref[...]), 0.0)
        h_s[pl.ds(base, bm), :feat] = h.astype(jnp.bfloat16)
        h_s[pl.ds(base, bm), feat:feat + 1] = jnp.ones((bm, 1), jnp.bfloat16)
        out_ref[...] = h

    @pl.when(l == 1)
    def _layer2():
        prod = jnp.dot(a, h_s[...], preferred_element_type=jnp.float32)
        xs2 = h_s[pl.ds(base, bm), :feat].astype(jnp.float32)
        out_ref[...] = _epilogue(prod, xs2)


def _pick_bm(n):
    for c in (200, 400, 256, 128, 80, 64, 40, 32, 16, 8):
        if n % c == 0:
            return c
    return n


def kernel(fts, adj, W1, b1, W2, b2):
    n, feat = fts.shape
    bm = _pick_bm(n)
    xa0 = jnp.concatenate(
        [fts.astype(jnp.bfloat16), jnp.ones((n, 1), jnp.bfloat16)], axis=1)
    ws = jnp.stack([W1[:feat], W2[:feat]])
    wn = jnp.stack([W1[feat:], W2[feat:]])
    bb = jnp.stack([b1.reshape(1, feat), b2.reshape(1, feat)])
    body = functools.partial(_fused_body, feat=feat, bm=bm)
    return pl.pallas_call(
        body,
        grid=(2, n // bm),
        in_specs=[
            pl.BlockSpec((bm, n), lambda l, i: (i, 0)),
            pl.BlockSpec((n, feat + 1), lambda l, i: (0, 0)),
            pl.BlockSpec((bm, feat), lambda l, i: (i, 0)),
            pl.BlockSpec((1, feat, feat), lambda l, i: (l, 0, 0)),
            pl.BlockSpec((1, feat, feat), lambda l, i: (l, 0, 0)),
            pl.BlockSpec((1, 1, feat), lambda l, i: (l, 0, 0)),
        ],
        out_specs=pl.BlockSpec((bm, feat), lambda l, i: (i, 0)),
        out_shape=jax.ShapeDtypeStruct((n, feat), jnp.float32),
        scratch_shapes=[pltpu.VMEM((n, feat + 1), jnp.bfloat16)],
        compiler_params=pltpu.CompilerParams(
            dimension_semantics=("arbitrary", "arbitrary"),
        ),
    )(adj, xa0, fts, ws, wn, bb)


# speedup vs baseline: 1.2115x; 1.2115x over previous
import functools

import jax
import jax.numpy as jnp
from jax.experimental import pallas as pl
from jax.experimental.pallas import tpu as pltpu


def _fused_body(adj_ref, xa0_ref, w_ref, out_ref, h_s, *, feat, bm):
    l = pl.program_id(0)
    i = pl.program_id(1)
    a = adj_ref[...].astype(jnp.bfloat16)
    base = pl.multiple_of(i * bm, bm)
    ws = w_ref[0, :feat]
    wn = w_ref[0, feat:2 * feat]
    b = w_ref[0, 2 * feat:2 * feat + 1]

    def _epilogue(prod, xs):
        s = prod[:, :feat]
        deg = jnp.clip(prod[:, feat:feat + 1], 1e-6, None)
        neigh = s / deg
        return (jnp.dot(xs, ws, preferred_element_type=jnp.float32,
                        precision=jax.lax.Precision.HIGHEST)
                + jnp.dot(neigh, wn, preferred_element_type=jnp.float32,
                          precision=jax.lax.Precision.HIGHEST)
                + b)

    @pl.when(l == 0)
    def _layer1():
        prod = jnp.dot(a, xa0_ref[...], preferred_element_type=jnp.float32)
        xs = xa0_ref[pl.ds(base, bm), :feat].astype(jnp.float32)
        h = jnp.maximum(_epilogue(prod, xs), 0.0)
        h_s[pl.ds(base, bm), :feat] = h.astype(jnp.bfloat16)
        h_s[pl.ds(base, bm), feat:feat + 1] = jnp.ones((bm, 1), jnp.bfloat16)
        out_ref[...] = h

    @pl.when(l == 1)
    def _layer2():
        prod = jnp.dot(a, h_s[...], preferred_element_type=jnp.float32)
        xs2 = h_s[pl.ds(base, bm), :feat].astype(jnp.float32)
        out_ref[...] = _epilogue(prod, xs2)


def _pick_bm(n):
    for c in (400, 256, 200, 128, 80, 64, 40, 32, 16, 8):
        if n % c == 0:
            return c
    return n


def kernel(fts, adj, W1, b1, W2, b2):
    n, feat = fts.shape
    bm = _pick_bm(n)
    xa0 = jnp.concatenate(
        [fts.astype(jnp.bfloat16), jnp.ones((n, 1), jnp.bfloat16)], axis=1)
    wpack = jnp.stack([
        jnp.concatenate([W1[:feat], W1[feat:], b1.reshape(1, feat)], axis=0),
        jnp.concatenate([W2[:feat], W2[feat:], b2.reshape(1, feat)], axis=0),
    ])
    body = functools.partial(_fused_body, feat=feat, bm=bm)
    return pl.pallas_call(
        body,
        grid=(2, n // bm),
        in_specs=[
            pl.BlockSpec((bm, n), lambda l, i: (i, 0)),
            pl.BlockSpec((n, feat + 1), lambda l, i: (0, 0)),
            pl.BlockSpec((1, 2 * feat + 1, feat), lambda l, i: (l, 0, 0)),
        ],
        out_specs=pl.BlockSpec((bm, feat), lambda l, i: (i, 0)),
        out_shape=jax.ShapeDtypeStruct((n, feat), jnp.float32),
        scratch_shapes=[pltpu.VMEM((n, feat + 1), jnp.bfloat16)],
        compiler_params=pltpu.CompilerParams(
            dimension_semantics=("arbitrary", "arbitrary"),
        ),
    )(adj, xa0, wpack)


# speedup vs baseline: 1.2183x; 1.0055x over previous
import functools

import jax
import jax.numpy as jnp
from jax.experimental import pallas as pl
from jax.experimental.pallas import tpu as pltpu


def _fused_body(adj_ref, xa0_ref, w_ref, out_ref, h_s, *, feat, bm):
    l = pl.program_id(0)
    i = pl.program_id(1)
    a = adj_ref[...].astype(jnp.bfloat16)
    base = pl.multiple_of(i * bm, bm)
    ws = w_ref[0, :feat]
    wn = w_ref[0, feat:2 * feat]
    b = w_ref[0, 2 * feat:2 * feat + 1]

    def _epilogue(prod, xs):
        s = prod[:, :feat]
        deg = jnp.clip(prod[:, feat:feat + 1], 1e-6, None)
        neigh = s / deg
        return (jnp.dot(xs, ws, preferred_element_type=jnp.float32,
                        precision=jax.lax.Precision.HIGHEST)
                + jnp.dot(neigh, wn, preferred_element_type=jnp.float32,
                          precision=jax.lax.Precision.HIGHEST)
                + b)

    @pl.when(l == 0)
    def _layer1():
        prod = jnp.dot(a, xa0_ref[...], preferred_element_type=jnp.float32)
        xs = xa0_ref[pl.ds(base, bm), :feat].astype(jnp.float32)
        h = jnp.maximum(_epilogue(prod, xs), 0.0)
        h_s[pl.ds(base, bm), :feat] = h.astype(jnp.bfloat16)
        h_s[pl.ds(base, bm), feat:feat + 1] = jnp.ones((bm, 1), jnp.bfloat16)
        out_ref[...] = h

    @pl.when(l == 1)
    def _layer2():
        prod = jnp.dot(a, h_s[...], preferred_element_type=jnp.float32)
        xs2 = h_s[pl.ds(base, bm), :feat].astype(jnp.float32)
        out_ref[...] = _epilogue(prod, xs2)


def _pick_bm(n):
    for c in (400, 256, 200, 128, 80, 64, 40, 32, 16, 8):
        if n % c == 0:
            return c
    return n


def kernel(fts, adj, W1, b1, W2, b2):
    n, feat = fts.shape
    bm = _pick_bm(n)
    xa0 = jnp.concatenate(
        [fts.astype(jnp.bfloat16), jnp.ones((n, 1), jnp.bfloat16)], axis=1)
    wpack = jnp.stack([
        jnp.concatenate([W1[:feat], W1[feat:], b1.reshape(1, feat)], axis=0),
        jnp.concatenate([W2[:feat], W2[feat:], b2.reshape(1, feat)], axis=0),
    ])
    body = functools.partial(_fused_body, feat=feat, bm=bm)
    return pl.pallas_call(
        body,
        grid=(2, n // bm),
        in_specs=[
            pl.BlockSpec((bm, n), lambda l, i: (i, 0)),
            pl.BlockSpec((n, feat + 1), lambda l, i: (0, 0)),
            pl.BlockSpec((1, 2 * feat + 1, feat), lambda l, i: (l, 0, 0)),
        ],
        out_specs=pl.BlockSpec((bm, feat), lambda l, i: (i * l, 0)),
        out_shape=jax.ShapeDtypeStruct((n, feat), jnp.float32),
        scratch_shapes=[pltpu.VMEM((n, feat + 1), jnp.bfloat16)],
        compiler_params=pltpu.CompilerParams(
            dimension_semantics=("arbitrary", "arbitrary"),
        ),
    )(adj, xa0, wpack)
